# baseline (speedup 1.0000x reference)
import functools

import jax
import jax.numpy as jnp
from jax.experimental import pallas as pl
from jax.experimental.pallas import tpu as pltpu

_MEAN_DIM = 4096
_EPS = 1e-05
_BLOCK_WROWS = 256
_STRIP = 8
_COL_CHUNK = 512
_UNROLL = 8

_MAG_MASK = 0x7FFF << 13
_REBIAS = 112 << 23
_SIGN_MASK = 0x80008000 - (1 << 32)


def _decode(bits):
    return pltpu.bitcast(bits + _REBIAS, jnp.float32)


def _encode(f):
    u = pltpu.bitcast(f, jnp.int32)
    return jnp.maximum((u + (0x1000 - _REBIAS)) >> 13, 0)


_SCALE_UP = float(2.0 ** 112)


def _one_strip(in_buf, out_buf, r, n_chunks):
    acc_lo = None
    acc_hi = None
    for c in range(n_chunks):
        cs = slice(c * _COL_CHUNK, (c + 1) * _COL_CHUNK)
        w = in_buf[r, cs]
        lo = _decode((w << 13) & _MAG_MASK)
        hi = _decode((w >> 3) & _MAG_MASK)
        sq_lo = lo * lo
        sq_hi = hi * hi
        acc_lo = sq_lo if acc_lo is None else acc_lo + sq_lo
        acc_hi = sq_hi if acc_hi is None else acc_hi + sq_hi
    s_lo = jax.lax.rsqrt(
        jnp.sum(acc_lo, axis=-1, keepdims=True) * (1.0 / _MEAN_DIM)
        + _EPS) * _SCALE_UP
    s_hi = jax.lax.rsqrt(
        jnp.sum(acc_hi, axis=-1, keepdims=True) * (1.0 / _MEAN_DIM)
        + _EPS) * _SCALE_UP
    for c in range(n_chunks):
        cs = slice(c * _COL_CHUNK, (c + 1) * _COL_CHUNK)
        w = in_buf[r, cs]
        lo = pltpu.bitcast((w << 13) & _MAG_MASK, jnp.float32)
        hi = pltpu.bitcast((w >> 3) & _MAG_MASK, jnp.float32)
        h_lo = _encode(lo * s_lo)
        h_hi = _encode(hi * s_hi)
        packed = pltpu.bitcast(
            pltpu.pack_elementwise([h_lo, h_hi], packed_dtype=jnp.int16),
            jnp.int32)
        out_buf[r, cs] = packed | (w & _SIGN_MASK)


def _process_block(in_buf, out_buf, cols):
    n_chunks = cols // _COL_CHUNK

    def strip2(i, _):
        for k in range(_UNROLL):
            _one_strip(in_buf, out_buf,
                       pl.ds((i * _UNROLL + k) * _STRIP, _STRIP), n_chunks)
        return 0

    jax.lax.fori_loop(0, _BLOCK_WROWS // (_UNROLL * _STRIP), strip2, 0)


def _rmsnorm_pipeline(x_hbm, o_hbm, in_buf, out_buf, in_sem, out_sem, *,
                      wrows, cols, n_blocks):
    xw = x_hbm.bitcast(jnp.int32)
    ow = o_hbm.bitcast(jnp.int32)
    step = pl.program_id(0)
    n_steps = n_blocks
    base = 0
    slot = jax.lax.rem(step, 2)
    nxt = jax.lax.rem(step + 1, 2)

    def in_copy(i, s):
        blk = pl.ds((base + i) * _BLOCK_WROWS, _BLOCK_WROWS)
        return pltpu.make_async_copy(xw.at[blk, :], in_buf.at[s], in_sem.at[s])

    def out_copy(i, s):
        blk = pl.ds((base + i) * _BLOCK_WROWS, _BLOCK_WROWS)
        return pltpu.make_async_copy(out_buf.at[s], ow.at[blk, :],
                                     out_sem.at[s])

    @pl.when(step == 0)
    def _():
        in_copy(0, 0).start()

    @pl.when(step + 1 < n_steps)
    def _():
        in_copy(step + 1, nxt).start()

    @pl.when(step >= 2)
    def _():
        out_copy(step - 2, slot).wait()

    in_copy(step, slot).wait()
    _process_block(in_buf.at[slot], out_buf.at[slot], cols)
    out_copy(step, slot).start()

    @pl.when(step == n_steps - 1)
    def _():
        out_copy(step, slot).wait()
        @pl.when(step >= 1)
        def _():
            out_copy(step - 1, nxt).wait()


def kernel(x):
    b, s, h = x.shape
    rows = b * s
    wrows = rows // 2
    n_blocks = wrows // _BLOCK_WROWS
    xb = jax.lax.bitcast_convert_type(
        x.reshape(-1), jnp.bfloat16).reshape(rows, h)
    ob = pl.pallas_call(
        functools.partial(_rmsnorm_pipeline, wrows=wrows, cols=h,
                          n_blocks=n_blocks),
        grid=(n_blocks,),
        in_specs=[pl.BlockSpec(memory_space=pltpu.MemorySpace.HBM)],
        out_specs=pl.BlockSpec(memory_space=pltpu.MemorySpace.HBM),
        out_shape=jax.ShapeDtypeStruct((rows, h), jnp.bfloat16),
        scratch_shapes=[
            pltpu.VMEM((2, _BLOCK_WROWS, h), jnp.int32),
            pltpu.VMEM((2, _BLOCK_WROWS, h), jnp.int32),
            pltpu.SemaphoreType.DMA((2,)),
            pltpu.SemaphoreType.DMA((2,)),
        ],
        compiler_params=pltpu.CompilerParams(
            dimension_semantics=("arbitrary",),
        ),
    )(xb)
    out = jax.lax.bitcast_convert_type(ob.reshape(-1), jnp.float16)
    return out.reshape(b, s, h)


# speedup vs baseline: 1.0052x; 1.0052x over previous
import functools

import jax
import jax.numpy as jnp
from jax.experimental import pallas as pl
from jax.experimental.pallas import tpu as pltpu

_MEAN_DIM = 4096
_EPS = 1e-05
_BLOCK_WROWS = 512
_STRIP = 8
_COL_CHUNK = 512
_UNROLL = 16

_MAG_MASK = 0x7FFF << 13
_REBIAS = 112 << 23
_SIGN_MASK = 0x80008000 - (1 << 32)


def _decode(bits):
    return pltpu.bitcast(bits + _REBIAS, jnp.float32)


def _encode(f):
    u = pltpu.bitcast(f, jnp.int32)
    return jnp.maximum((u + (0x1000 - _REBIAS)) >> 13, 0)


_SCALE_UP = float(2.0 ** 112)


def _one_strip(in_buf, out_buf, r, n_chunks):
    acc_lo = None
    acc_hi = None
    for c in range(n_chunks):
        cs = slice(c * _COL_CHUNK, (c + 1) * _COL_CHUNK)
        w = in_buf[r, cs]
        lo = _decode((w << 13) & _MAG_MASK)
        hi = _decode((w >> 3) & _MAG_MASK)
        sq_lo = lo * lo
        sq_hi = hi * hi
        acc_lo = sq_lo if acc_lo is None else acc_lo + sq_lo
        acc_hi = sq_hi if acc_hi is None else acc_hi + sq_hi
    s_lo = jax.lax.rsqrt(
        jnp.sum(acc_lo, axis=-1, keepdims=True) * (1.0 / _MEAN_DIM)
        + _EPS) * _SCALE_UP
    s_hi = jax.lax.rsqrt(
        jnp.sum(acc_hi, axis=-1, keepdims=True) * (1.0 / _MEAN_DIM)
        + _EPS) * _SCALE_UP
    for c in range(n_chunks):
        cs = slice(c * _COL_CHUNK, (c + 1) * _COL_CHUNK)
        w = in_buf[r, cs]
        lo = pltpu.bitcast((w << 13) & _MAG_MASK, jnp.float32)
        hi = pltpu.bitcast((w >> 3) & _MAG_MASK, jnp.float32)
        h_lo = _encode(lo * s_lo)
        h_hi = _encode(hi * s_hi)
        packed = pltpu.bitcast(
            pltpu.pack_elementwise([h_lo, h_hi], packed_dtype=jnp.int16),
            jnp.int32)
        out_buf[r, cs] = packed | (w & _SIGN_MASK)


def _process_block(in_buf, out_buf, cols):
    n_chunks = cols // _COL_CHUNK

    def strip2(i, _):
        for k in range(_UNROLL):
            _one_strip(in_buf, out_buf,
                       pl.ds((i * _UNROLL + k) * _STRIP, _STRIP), n_chunks)
        return 0

    jax.lax.fori_loop(0, _BLOCK_WROWS // (_UNROLL * _STRIP), strip2, 0)


def _rmsnorm_pipeline(x_hbm, o_hbm, in_buf, out_buf, in_sem, out_sem, *,
                      wrows, cols, n_blocks):
    xw = x_hbm.bitcast(jnp.int32)
    ow = o_hbm.bitcast(jnp.int32)
    step = pl.program_id(0)
    n_steps = n_blocks
    base = 0
    slot = jax.lax.rem(step, 2)
    nxt = jax.lax.rem(step + 1, 2)

    def in_copy(i, s):
        blk = pl.ds((base + i) * _BLOCK_WROWS, _BLOCK_WROWS)
        return pltpu.make_async_copy(xw.at[blk, :], in_buf.at[s], in_sem.at[s])

    def out_copy(i, s):
        blk = pl.ds((base + i) * _BLOCK_WROWS, _BLOCK_WROWS)
        return pltpu.make_async_copy(out_buf.at[s], ow.at[blk, :],
                                     out_sem.at[s])

    @pl.when(step == 0)
    def _():
        in_copy(0, 0).start()

    @pl.when(step + 1 < n_steps)
    def _():
        in_copy(step + 1, nxt).start()

    @pl.when(step >= 2)
    def _():
        out_copy(step - 2, slot).wait()

    in_copy(step, slot).wait()
    _process_block(in_buf.at[slot], out_buf.at[slot], cols)
    out_copy(step, slot).start()

    @pl.when(step == n_steps - 1)
    def _():
        out_copy(step, slot).wait()
        @pl.when(step >= 1)
        def _():
            out_copy(step - 1, nxt).wait()


def kernel(x):
    b, s, h = x.shape
    rows = b * s
    wrows = rows // 2
    n_blocks = wrows // _BLOCK_WROWS
    xb = jax.lax.bitcast_convert_type(
        x.reshape(-1), jnp.bfloat16).reshape(rows, h)
    ob = pl.pallas_call(
        functools.partial(_rmsnorm_pipeline, wrows=wrows, cols=h,
                          n_blocks=n_blocks),
        grid=(n_blocks,),
        in_specs=[pl.BlockSpec(memory_space=pltpu.MemorySpace.HBM)],
        out_specs=pl.BlockSpec(memory_space=pltpu.MemorySpace.HBM),
        out_shape=jax.ShapeDtypeStruct((rows, h), jnp.bfloat16),
        scratch_shapes=[
            pltpu.VMEM((2, _BLOCK_WROWS, h), jnp.int32),
            pltpu.VMEM((2, _BLOCK_WROWS, h), jnp.int32),
            pltpu.SemaphoreType.DMA((2,)),
            pltpu.SemaphoreType.DMA((2,)),
        ],
        compiler_params=pltpu.CompilerParams(
            dimension_semantics=("arbitrary",),
        ),
    )(xb)
    out = jax.lax.bitcast_convert_type(ob.reshape(-1), jnp.float16)
    return out.reshape(b, s, h)
